# baseline (speedup 1.0000x reference)
import functools

import jax
import jax.numpy as jnp
from jax import lax
from jax.experimental import pallas as pl
from jax.experimental.pallas import tpu as pltpu
from jax.experimental.pallas import tpu_sc as plsc

N_TIME_INTERVAL = 64
TIME_DIM = 32
PASS_TIME = 1000000.0
PER_TIME = PASS_TIME / N_TIME_INTERVAL
INV_PER_TIME = 1.0 / PER_TIME

NUM_CORES = 2
NUM_SUBCORES = 16
NUM_WORKERS = NUM_CORES * NUM_SUBCORES

BATCH = 4096
MAX_LEN = 200
TS_LEN = MAX_LEN + 1
BPW = BATCH // NUM_WORKERS
RPW = BPW * MAX_LEN
SLEN = 400
NBLOCK = RPW // SLEN
NBUF = 4
LANES = 16
TPAD = TIME_DIM + 1
DIV_MUL = 5243
DIV_SHIFT = 20


def _sc_body(ts_hbm, table_hbm, out_hbm, ts_v, table_v, obufs, sems):
    wid = lax.axis_index("s") * NUM_CORES + lax.axis_index("c")

    pltpu.sync_copy(ts_hbm.at[pl.ds(wid * (BPW * TS_LEN), BPW * TS_LEN)], ts_v)
    pltpu.sync_copy(table_hbm, table_v)

    lane = lax.iota(jnp.int32, LANES)
    swz = [jnp.bitwise_and(lane + c, TIME_DIM - 1) for c in range(TIME_DIM)]

    def out_dma(j, b):
        return pltpu.make_async_copy(
            obufs.at[b],
            out_hbm.at[pl.ds(wid * RPW + j * SLEN, SLEN)],
            sems.at[b])

    def block_body(j0, carry):
        for b in range(NBUF):
            j = j0 * NBUF + b

            @pl.when(j >= NBUF)
            def _():
                out_dma(j - NBUF, b).wait()

            obuf = obufs.at[b]
            jbase = j * SLEN

            @plsc.parallel_loop(0, SLEN, LANES, unroll=4)
            def _(r):
                p = lane + (jbase + r)
                bp = jnp.right_shift(p * DIV_MUL, DIV_SHIFT)
                o = p + bp
                t_lo = plsc.load_gather(ts_v, [o])
                t_hi = plsc.load_gather(ts_v, [o + 1])
                delta = (t_hi - t_lo) * INV_PER_TIME
                vi = lax.convert_element_type(delta, jnp.int32)
                vi = jnp.minimum(jnp.maximum(vi, 0), N_TIME_INTERVAL - 1)
                base = vi * TPAD
                row = lane + r
                for c0 in range(0, TIME_DIM, 8):
                    vals = [plsc.load_gather(table_v, [base + swz[c0 + k]])
                            for k in range(8)]
                    for k in range(8):
                        plsc.store_scatter(obuf, [row, swz[c0 + k]], vals[k])

            out_dma(j, b).start()
        return carry

    lax.fori_loop(0, NBLOCK // NBUF, block_body, 0)

    for b in range(NBUF):
        out_dma(NBLOCK - NBUF + b, b).wait()


@functools.partial(
    pl.kernel,
    mesh=plsc.VectorSubcoreMesh(core_axis_name="c", subcore_axis_name="s"),
    out_type=jax.ShapeDtypeStruct((BATCH * MAX_LEN, TIME_DIM), jnp.float32),
    scratch_types=[
        pltpu.VMEM((BPW * TS_LEN,), jnp.float32),
        pltpu.VMEM((N_TIME_INTERVAL * TPAD,), jnp.float32),
        pltpu.VMEM((NBUF, SLEN, TIME_DIM), jnp.float32),
        pltpu.SemaphoreType.DMA((NBUF,)),
    ],
    compiler_params=pltpu.CompilerParams(
        use_tc_tiling_on_sc=False, needs_layout_passes=False),
)
def _time_encode_sc(ts_hbm, table_hbm, out_hbm, ts_v, table_v, obufs, sems):
    _sc_body(ts_hbm, table_hbm, out_hbm, ts_v, table_v, obufs, sems)


def kernel(input, timestamp, train, W, b):
    batch_size, max_len = input.shape
    table = jnp.pad(W.T + b[None, :], ((0, 0), (0, 1))).reshape(-1)
    flat = _time_encode_sc(timestamp.reshape(-1), table)
    time_embedding = flat.reshape(batch_size, max_len, TIME_DIM)
    return (time_embedding, timestamp[:, :-1])


# speedup vs baseline: 1.6558x; 1.6558x over previous
import functools

import jax
import jax.numpy as jnp
from jax import lax
from jax.experimental import pallas as pl
from jax.experimental.pallas import tpu as pltpu
from jax.experimental.pallas import tpu_sc as plsc

N_TIME_INTERVAL = 64
TIME_DIM = 32
PASS_TIME = 1000000.0
PER_TIME = PASS_TIME / N_TIME_INTERVAL
INV_PER_TIME = 1.0 / PER_TIME

NUM_CORES = 2
NUM_SUBCORES = 16
NUM_WORKERS = NUM_CORES * NUM_SUBCORES

BATCH = 4096
MAX_LEN = 200
TS_LEN = MAX_LEN + 1
BPW = BATCH // NUM_WORKERS
RPW = BPW * MAX_LEN
SLEN = 320
NBLOCK = RPW // SLEN
NBUF = 2
LANES = 16
TPAD = TIME_DIM + 1
DIV_MUL = 5243
DIV_SHIFT = 20


def _sc_body(ts_hbm, table_hbm, out_hbm, ts_v, table_v, obufs, sems):
    wid = lax.axis_index("s") * NUM_CORES + lax.axis_index("c")

    pltpu.sync_copy(ts_hbm.at[pl.ds(wid * (BPW * TS_LEN), BPW * TS_LEN)], ts_v)
    pltpu.sync_copy(table_hbm, table_v)

    lane = lax.iota(jnp.int32, LANES)
    swz = [jnp.bitwise_and(lane + c, TIME_DIM - 1) for c in range(TIME_DIM)]

    def out_dma(j, b):
        return pltpu.make_async_copy(
            obufs.at[b],
            out_hbm.at[pl.ds(wid * RPW + j * SLEN, SLEN)],
            sems.at[b])

    def block_body(j0, carry):
        for b in range(NBUF):
            j = j0 * NBUF + b

            @pl.when(j >= NBUF)
            def _():
                out_dma(j - NBUF, b).wait()

            obuf = obufs.at[b]
            jbase = j * SLEN

            @plsc.parallel_loop(0, SLEN, LANES, unroll=4)
            def _(r):
                p = lane + (jbase + r)
                bp = jnp.right_shift(p * DIV_MUL, DIV_SHIFT)
                o = p + bp
                t_lo = plsc.load_gather(ts_v, [o])
                t_hi = plsc.load_gather(ts_v, [o + 1])
                delta = (t_hi - t_lo) * INV_PER_TIME
                vi = lax.convert_element_type(delta, jnp.int32)
                vi = jnp.minimum(jnp.maximum(vi, 0), N_TIME_INTERVAL - 1)
                base = vi * TPAD
                row = lane + r
                for c0 in range(0, TIME_DIM, 8):
                    vals = [plsc.load_gather(table_v, [base + swz[c0 + k]])
                            for k in range(8)]
                    for k in range(8):
                        plsc.store_scatter(obuf, [row, swz[c0 + k]], vals[k])

            out_dma(j, b).start()
        return carry

    lax.fori_loop(0, NBLOCK // NBUF, block_body, 0)

    for b in range(NBUF):
        out_dma(NBLOCK - NBUF + b, b).wait()


@functools.partial(
    pl.kernel,
    mesh=plsc.VectorSubcoreMesh(core_axis_name="c", subcore_axis_name="s"),
    out_type=jax.ShapeDtypeStruct((BATCH * MAX_LEN, TIME_DIM), jnp.float32),
    scratch_types=[
        pltpu.VMEM((BPW * TS_LEN,), jnp.float32),
        pltpu.VMEM((2176,), jnp.float32),
        pltpu.VMEM((NBUF, SLEN, TIME_DIM), jnp.float32),
        pltpu.SemaphoreType.DMA((NBUF,)),
    ],
    compiler_params=pltpu.CompilerParams(
        use_tc_tiling_on_sc=True, needs_layout_passes=False),
)
def _time_encode_sc(ts_hbm, table_hbm, out_hbm, ts_v, table_v, obufs, sems):
    _sc_body(ts_hbm, table_hbm, out_hbm, ts_v, table_v, obufs, sems)


def kernel(input, timestamp, train, W, b):
    batch_size, max_len = input.shape
    table = jnp.pad(W.T + b[None, :], ((0, 0), (0, 1))).reshape(-1)
    table = jnp.pad(table, (0, 2176 - table.shape[0]))
    flat = _time_encode_sc(timestamp.reshape(-1), table)
    time_embedding = flat.reshape(batch_size, max_len, TIME_DIM)
    return (time_embedding, timestamp[:, :-1])


# speedup vs baseline: 1.7626x; 1.0645x over previous
import functools

import jax
import jax.numpy as jnp
from jax import lax
from jax.experimental import pallas as pl
from jax.experimental.pallas import tpu as pltpu
from jax.experimental.pallas import tpu_sc as plsc

N_TIME_INTERVAL = 64
TIME_DIM = 32
PASS_TIME = 1000000.0
PER_TIME = PASS_TIME / N_TIME_INTERVAL
INV_PER_TIME = 1.0 / PER_TIME

NUM_CORES = 2
NUM_SUBCORES = 16
NUM_WORKERS = NUM_CORES * NUM_SUBCORES

BATCH = 4096
MAX_LEN = 200
TS_LEN = MAX_LEN + 1
BPW = BATCH // NUM_WORKERS
RPW = BPW * MAX_LEN
SLEN = 320
NBLOCK = RPW // SLEN
NBUF = 2
LANES = 16
TPAD = TIME_DIM + 1
DIV_MUL = 5243
DIV_SHIFT = 20


def _sc_body(ts_hbm, table_hbm, out_hbm, ts_v, table_v, obufs, sems):
    wid = lax.axis_index("s") * NUM_CORES + lax.axis_index("c")

    pltpu.sync_copy(ts_hbm.at[pl.ds(wid * BPW, BPW)], ts_v)
    pltpu.sync_copy(table_hbm, table_v)

    lane = lax.iota(jnp.int32, LANES)
    swz = [jnp.bitwise_and(lane + c, TIME_DIM - 1) for c in range(TIME_DIM)]

    def out_dma(j, b):
        return pltpu.make_async_copy(
            obufs.at[b],
            out_hbm.at[pl.ds(wid * RPW + j * SLEN, SLEN)],
            sems.at[b])

    def block_body(j0, carry):
        for b in range(NBUF):
            j = j0 * NBUF + b

            @pl.when(j >= NBUF)
            def _():
                out_dma(j - NBUF, b).wait()

            obuf = obufs.at[b]
            jbase = j * SLEN

            @plsc.parallel_loop(0, SLEN, LANES, unroll=4)
            def _(r):
                p = lane + (jbase + r)
                bp = jnp.right_shift(p * DIV_MUL, DIV_SHIFT)
                col = p - bp * MAX_LEN
                t_lo = plsc.load_gather(ts_v, [bp, col])
                t_hi = plsc.load_gather(ts_v, [bp, col + 1])
                delta = (t_hi - t_lo) * INV_PER_TIME
                vi = lax.convert_element_type(delta, jnp.int32)
                vi = jnp.minimum(jnp.maximum(vi, 0), N_TIME_INTERVAL - 1)
                base = vi * TPAD
                row = lane + r
                for c0 in range(0, TIME_DIM, 8):
                    vals = [plsc.load_gather(table_v, [base + swz[c0 + k]])
                            for k in range(8)]
                    for k in range(8):
                        plsc.store_scatter(obuf, [row, swz[c0 + k]], vals[k])

            out_dma(j, b).start()
        return carry

    lax.fori_loop(0, NBLOCK // NBUF, block_body, 0)

    for b in range(NBUF):
        out_dma(NBLOCK - NBUF + b, b).wait()


@functools.partial(
    pl.kernel,
    mesh=plsc.VectorSubcoreMesh(core_axis_name="c", subcore_axis_name="s"),
    out_type=jax.ShapeDtypeStruct((BATCH * MAX_LEN, TIME_DIM), jnp.float32),
    scratch_types=[
        pltpu.VMEM((BPW, TS_LEN), jnp.float32),
        pltpu.VMEM((2176,), jnp.float32),
        pltpu.VMEM((NBUF, SLEN, TIME_DIM), jnp.float32),
        pltpu.SemaphoreType.DMA((NBUF,)),
    ],
    compiler_params=pltpu.CompilerParams(
        use_tc_tiling_on_sc=True, needs_layout_passes=False),
)
def _time_encode_sc(ts_hbm, table_hbm, out_hbm, ts_v, table_v, obufs, sems):
    _sc_body(ts_hbm, table_hbm, out_hbm, ts_v, table_v, obufs, sems)


def kernel(input, timestamp, train, W, b):
    batch_size, max_len = input.shape
    table = jnp.pad(W.T + b[None, :], ((0, 0), (0, 1))).reshape(-1)
    table = jnp.pad(table, (0, 2176 - table.shape[0]))
    flat = _time_encode_sc(timestamp, table)
    time_embedding = flat.reshape(batch_size, max_len, TIME_DIM)
    return (time_embedding, timestamp[:, :-1])


# speedup vs baseline: 1.8022x; 1.0225x over previous
import functools

import jax
import jax.numpy as jnp
from jax import lax
from jax.experimental import pallas as pl
from jax.experimental.pallas import tpu as pltpu
from jax.experimental.pallas import tpu_sc as plsc

N_TIME_INTERVAL = 64
TIME_DIM = 32
PASS_TIME = 1000000.0
PER_TIME = PASS_TIME / N_TIME_INTERVAL
INV_PER_TIME = 1.0 / PER_TIME

NUM_CORES = 2
NUM_SUBCORES = 16
NUM_WORKERS = NUM_CORES * NUM_SUBCORES

BATCH = 4096
MAX_LEN = 200
TS_LEN = MAX_LEN + 1
BPW = BATCH // NUM_WORKERS
RPW = BPW * MAX_LEN
SLEN = 320
NBLOCK = RPW // SLEN
NBUF = 2
LANES = 16
TPAD = TIME_DIM + 1
DIV_MUL = 5243
DIV_SHIFT = 20
BPG = (8 * MAX_LEN) // SLEN
PBROWS = 8
NGROUPS = BPW // PBROWS


def _sc_body(ts_hbm, table_hbm, out_hbm, pass_hbm, ts_v, table_v, obufs, pbufs,
             sems, psems):
    wid = lax.axis_index("s") * NUM_CORES + lax.axis_index("c")

    pltpu.sync_copy(ts_hbm.at[pl.ds(wid * BPW, BPW)], ts_v)
    pltpu.sync_copy(table_hbm, table_v)

    lane = lax.iota(jnp.int32, LANES)
    swz = [jnp.bitwise_and(lane + c, TIME_DIM - 1) for c in range(TIME_DIM)]

    def out_dma(j, b):
        return pltpu.make_async_copy(
            obufs.at[b],
            out_hbm.at[pl.ds(wid * RPW + j * SLEN, SLEN)],
            sems.at[b])

    def pass_dma(g, q):
        return pltpu.make_async_copy(
            pbufs.at[pl.ds(q * PBROWS, PBROWS)],
            pass_hbm.at[pl.ds(wid * BPW + g * PBROWS, PBROWS)],
            psems.at[q])

    def block_body(j0, carry):
        for b in range(NBUF):
            j = j0 * NBUF + b
            g = lax.div(j, BPG)
            q = lax.rem(g, 2)
            m = lax.rem(j, BPG)

            @pl.when(j >= NBUF)
            def _():
                out_dma(j - NBUF, b).wait()

            @pl.when(jnp.logical_and(m == 0, g >= 2))
            def _():
                pass_dma(g - 2, q).wait()

            obuf = obufs.at[b]
            jbase = j * SLEN
            prow_off = (g - q) * PBROWS

            @plsc.parallel_loop(0, SLEN, LANES, unroll=4)
            def _(r):
                p = lane + (jbase + r)
                bp = jnp.right_shift(p * DIV_MUL, DIV_SHIFT)
                col = p - bp * MAX_LEN
                t_lo = plsc.load_gather(ts_v, [bp, col])
                t_hi = plsc.load_gather(ts_v, [bp, col + 1])
                plsc.store_scatter(pbufs, [bp - prow_off, col], t_lo)
                delta = (t_hi - t_lo) * INV_PER_TIME
                vi = lax.convert_element_type(delta, jnp.int32)
                vi = jnp.minimum(jnp.maximum(vi, 0), N_TIME_INTERVAL - 1)
                base = vi * TPAD
                row = lane + r
                for c0 in range(0, TIME_DIM, 8):
                    vals = [plsc.load_gather(table_v, [base + swz[c0 + k]])
                            for k in range(8)]
                    for k in range(8):
                        plsc.store_scatter(obuf, [row, swz[c0 + k]], vals[k])

            out_dma(j, b).start()

            @pl.when(m == BPG - 1)
            def _():
                pass_dma(g, q).start()
        return carry

    lax.fori_loop(0, NBLOCK // NBUF, block_body, 0)

    for b in range(NBUF):
        out_dma(NBLOCK - NBUF + b, b).wait()
    pass_dma(NGROUPS - 2, 0).wait()
    pass_dma(NGROUPS - 1, 1).wait()


@functools.partial(
    pl.kernel,
    mesh=plsc.VectorSubcoreMesh(core_axis_name="c", subcore_axis_name="s"),
    out_type=[
        jax.ShapeDtypeStruct((BATCH * MAX_LEN, TIME_DIM), jnp.float32),
        jax.ShapeDtypeStruct((BATCH, MAX_LEN), jnp.float32),
    ],
    scratch_types=[
        pltpu.VMEM((BPW, TS_LEN), jnp.float32),
        pltpu.VMEM((2176,), jnp.float32),
        pltpu.VMEM((NBUF, SLEN, TIME_DIM), jnp.float32),
        pltpu.VMEM((2 * PBROWS, MAX_LEN), jnp.float32),
        pltpu.SemaphoreType.DMA((NBUF,)),
        pltpu.SemaphoreType.DMA((2,)),
    ],
    compiler_params=pltpu.CompilerParams(
        use_tc_tiling_on_sc=True, needs_layout_passes=False),
)
def _time_encode_sc(ts_hbm, table_hbm, out_hbm, pass_hbm, ts_v, table_v,
                    obufs, pbufs, sems, psems):
    _sc_body(ts_hbm, table_hbm, out_hbm, pass_hbm, ts_v, table_v, obufs, pbufs,
             sems, psems)


def kernel(input, timestamp, train, W, b):
    batch_size, max_len = input.shape
    table = jnp.pad(W.T + b[None, :], ((0, 0), (0, 1))).reshape(-1)
    table = jnp.pad(table, (0, 2176 - table.shape[0]))
    flat, ts_pass = _time_encode_sc(timestamp, table)
    time_embedding = flat.reshape(batch_size, max_len, TIME_DIM)
    return (time_embedding, ts_pass)
